# Initial kernel scaffold; baseline (speedup 1.0000x reference)
import jax
import jax.numpy as jnp
from jax.experimental import pallas as pl


def kernel(x, edge_index, edge_type, W0, root0, b0, bn_gamma, bn_beta, bn_mean, bn_var, W1, root1, b1):
    raise NotImplementedError("write your pallas kernel here")


# speedup vs baseline: 6.0601x; 6.0601x over previous
import functools

import jax
import jax.numpy as jnp
from jax import lax
from jax.experimental import pallas as pl
from jax.experimental.pallas import tpu as pltpu
from jax.experimental.pallas import tpu_sc as plsc

N = 10000
NP = 10240
E = 320000
EP = 321536
R = 4
D = 128
CW = 32
NCHUNK = 4
IDX_ROWS = EP // 128
RPS = IDX_ROWS // 16
ACC_ROWS = R * NP + 16
SLICE = ACC_ROWS // 16
BN_EPS = 1e-5
BLK = 512


def _sc_edge_pass(table, gidx, sidx, zeros, zeros16, ones):
    mesh = plsc.VectorSubcoreMesh(core_axis_name="c", subcore_axis_name="s",
                                  num_cores=2, num_subcores=16)

    def body(table_ref, gidx_ref, sidx_ref, zeros_ref, zeros16_ref, ones_ref,
             acc_out, cnt_out, idxg, idxd, rowbuf, onesv, accsh, cntsh, sem):
        cid = lax.axis_index("c")
        sid = lax.axis_index("s")
        pltpu.sync_copy(ones_ref, onesv)
        for ci in range(2):
            chunk = cid * 2 + ci
            pltpu.sync_copy(zeros_ref, accsh.at[pl.ds(sid * SLICE, SLICE)])
            if ci == 0:
                pltpu.sync_copy(zeros16_ref,
                                cntsh.at[pl.ds(sid * SLICE, SLICE)])
            plsc.subcore_barrier()

            def step(i, carry):
                row = sid * RPS + i
                pltpu.sync_copy(gidx_ref.at[chunk, row], idxg)
                pltpu.sync_copy(sidx_ref.at[row], idxd)
                pltpu.async_copy(table_ref.at[idxg], rowbuf, sem).wait()
                pltpu.sync_copy(rowbuf, accsh.at[idxd], add=True)
                if ci == 0:
                    pltpu.sync_copy(onesv, cntsh.at[idxd], add=True)
                return carry

            lax.fori_loop(0, RPS, step, 0)
            plsc.subcore_barrier()
            pltpu.sync_copy(accsh.at[pl.ds(sid * SLICE, SLICE)],
                            acc_out.at[pl.ds(sid * SLICE, SLICE),
                                       pl.ds(chunk * CW, CW)])
            if ci == 0:
                pltpu.sync_copy(cntsh.at[pl.ds(sid * SLICE, SLICE)],
                                cnt_out.at[cid, pl.ds(sid * SLICE, SLICE)])

    run = pl.kernel(
        body,
        out_type=(
            jax.ShapeDtypeStruct((ACC_ROWS, D), jnp.float32),
            jax.ShapeDtypeStruct((2, ACC_ROWS, 16), jnp.float32),
        ),
        mesh=mesh,
        scratch_types=(
            pltpu.VMEM((128,), jnp.int32),
            pltpu.VMEM((128,), jnp.int32),
            pltpu.VMEM((128, CW), jnp.float32),
            pltpu.VMEM((128, 16), jnp.float32),
            pltpu.VMEM_SHARED((ACC_ROWS, CW), jnp.float32),
            pltpu.VMEM_SHARED((ACC_ROWS, 16), jnp.float32),
            pltpu.SemaphoreType.DMA,
        ),
        compiler_params=pltpu.CompilerParams(use_tc_tiling_on_sc=False),
    )
    return run(table, gidx, sidx, zeros, zeros16, ones)


def _tc_dense(xp, accf, cnt, wstack, gs, bs, relu):
    def body(x_ref, acc_ref, cnt_ref, w_ref, gs_ref, bs_ref, o_ref):
        inv = 1.0 / jnp.maximum(cnt_ref[...], 1.0)
        invb = lax.broadcast_in_dim(inv, (R, BLK, D), (0, 1))
        scaled = acc_ref[...] * invb
        parts = [x_ref[...]] + [scaled[r] for r in range(R)]
        inp = jnp.concatenate(parts, axis=1)
        out = jnp.dot(inp, w_ref[...], preferred_element_type=jnp.float32)
        out = out * gs_ref[...] + bs_ref[...]
        if relu:
            out = jnp.maximum(out, 0.0)
        o_ref[...] = out

    grid = (NP // BLK,)
    return pl.pallas_call(
        body,
        grid=grid,
        in_specs=[
            pl.BlockSpec((BLK, D), lambda i: (i, 0)),
            pl.BlockSpec((R, BLK, D), lambda i: (0, i, 0)),
            pl.BlockSpec((R, BLK), lambda i: (0, i)),
            pl.BlockSpec((5 * D, D), lambda i: (0, 0)),
            pl.BlockSpec((1, D), lambda i: (0, 0)),
            pl.BlockSpec((1, D), lambda i: (0, 0)),
        ],
        out_specs=pl.BlockSpec((BLK, D), lambda i: (i, 0)),
        out_shape=jax.ShapeDtypeStruct((NP, D), jnp.float32),
    )(xp, accf, cnt, wstack, gs, bs)


def kernel(x, edge_index, edge_type, W0, root0, b0, bn_gamma, bn_beta,
           bn_mean, bn_var, W1, root1, b1):
    f32 = jnp.float32
    src = edge_index[0].astype(jnp.int32)
    dst = edge_index[1].astype(jnp.int32)
    et = edge_type.astype(jnp.int32)

    base = jnp.pad(src * 4, (0, EP - E))
    gidx = (base[None, :] + jnp.arange(NCHUNK, dtype=jnp.int32)[:, None]
            ).reshape(NCHUNK, IDX_ROWS, 128)
    sflat = jnp.pad(et * NP + dst, (0, EP - E), constant_values=R * NP)
    sidx = sflat.reshape(IDX_ROWS, 128)

    zeros = jnp.zeros((SLICE, CW), f32)
    zeros16 = jnp.zeros((SLICE, 16), f32)
    ones = jnp.ones((128, 16), f32)

    xp = jnp.pad(x, ((0, NP - N), (0, 0)))

    gp = bn_gamma / jnp.sqrt(bn_var + BN_EPS)
    gs0 = gp.reshape(1, D)
    bs0 = ((b0 - bn_mean) * gp + bn_beta).reshape(1, D)
    gs1 = jnp.ones((1, D), f32)
    bs1 = b1.reshape(1, D)
    wstack0 = jnp.concatenate([root0] + [W0[r] for r in range(R)], axis=0)
    wstack1 = jnp.concatenate([root1] + [W1[r] for r in range(R)], axis=0)

    acc0, cnt2 = _sc_edge_pass(xp.reshape(NCHUNK * NP, CW),
                               gidx, sidx, zeros, zeros16, ones)
    cnt = cnt2[0, :R * NP, 0].reshape(R, NP)
    h = _tc_dense(xp, acc0[:R * NP].reshape(R, NP, D), cnt,
                  wstack0, gs0, bs0, relu=True)
    acc1, _ = _sc_edge_pass(h.reshape(NCHUNK * NP, CW),
                            gidx, sidx, zeros, zeros16, ones)
    out = _tc_dense(h, acc1[:R * NP].reshape(R, NP, D), cnt,
                    wstack1, gs1, bs1, relu=False)
    return out[:N]


# speedup vs baseline: 8.7308x; 1.4407x over previous
import functools

import jax
import jax.numpy as jnp
from jax import lax
from jax.experimental import pallas as pl
from jax.experimental.pallas import tpu as pltpu
from jax.experimental.pallas import tpu_sc as plsc

N = 10000
NP = 10240
E = 320000
R = 4
D = 128
CW = 32
NCHUNK = 4
KG = 4
G = 40
KGC = 8
GC = 10
T = G // 2
EP = 16 * G * KG * 128
RPS = KG * G
ACC_ROWS = R * NP + 16
SLICE = ACC_ROWS // 16
BN_EPS = 1e-5
BLK = 512


def _sc_edge_pass(table, pidx, zeros):
    mesh = plsc.VectorSubcoreMesh(core_axis_name="c", subcore_axis_name="s",
                                  num_cores=2, num_subcores=16)

    def body(table_ref, pidx_ref, zeros_ref,
             acc_out, idxb0, idxb1, rowb0, rowb1,
             accsh, semg0, semg1, semi):
        cid = lax.axis_index("c")
        sid = lax.axis_index("s")
        for ci in range(2):
            chunk = cid * 2 + ci
            pltpu.sync_copy(zeros_ref, accsh.at[pl.ds(sid * SLICE, SLICE)])
            plsc.subcore_barrier()

            gbase = sid * G

            def fire_gathers(idxb, rowb, sem):
                for j in range(KG):
                    pltpu.async_copy(table_ref.at[idxb.at[j]],
                                     rowb.at[pl.ds(j * 128, 128)], sem)

            def drain_gathers(rowb, sem):
                pltpu.make_async_copy(table_ref.at[pl.ds(0, KG * 128)],
                                      rowb, sem).wait()

            def load_idx(g, idxb):
                pltpu.async_copy(pidx_ref.at[chunk, gbase + g], idxb, semi)

            def wait_idx(idxb):
                pltpu.make_async_copy(pidx_ref.at[chunk, gbase], idxb,
                                      semi).wait()

            def scatter(idxb, rowb):
                for j in range(KG):
                    pltpu.sync_copy(rowb.at[pl.ds(j * 128, 128)],
                                    accsh.at[idxb.at[KG + j]], add=True)

            pltpu.sync_copy(pidx_ref.at[chunk, gbase], idxb0)
            fire_gathers(idxb0, rowb0, semg0)
            load_idx(1, idxb1)

            def step(t, carry):
                wait_idx(idxb1)
                fire_gathers(idxb1, rowb1, semg1)
                drain_gathers(rowb0, semg0)
                scatter(idxb0, rowb0)
                pl.when(t < T - 1)(lambda: load_idx(2 * t + 2, idxb0))

                def refill_a():
                    wait_idx(idxb0)
                    fire_gathers(idxb0, rowb0, semg0)
                pl.when(t < T - 1)(refill_a)

                drain_gathers(rowb1, semg1)
                scatter(idxb1, rowb1)
                pl.when(t < T - 1)(lambda: load_idx(2 * t + 3, idxb1))
                return carry

            lax.fori_loop(0, T, step, 0)
            plsc.subcore_barrier()
            pltpu.sync_copy(accsh.at[pl.ds(sid * SLICE, SLICE)],
                            acc_out.at[pl.ds(sid * SLICE, SLICE),
                                       pl.ds(chunk * CW, CW)])

    run = pl.kernel(
        body,
        out_type=jax.ShapeDtypeStruct((ACC_ROWS, D), jnp.float32),
        mesh=mesh,
        scratch_types=(
            pltpu.VMEM((2 * KG, 128), jnp.int32),
            pltpu.VMEM((2 * KG, 128), jnp.int32),
            pltpu.VMEM((KG * 128, CW), jnp.float32),
            pltpu.VMEM((KG * 128, CW), jnp.float32),
            pltpu.VMEM_SHARED((ACC_ROWS, CW), jnp.float32),
            pltpu.SemaphoreType.DMA,
            pltpu.SemaphoreType.DMA,
            pltpu.SemaphoreType.DMA,
        ),
        compiler_params=pltpu.CompilerParams(use_tc_tiling_on_sc=False),
    )
    return run(table, pidx, zeros)


def _sc_count_pass(cidx, zeros16, ones):
    mesh = plsc.VectorSubcoreMesh(core_axis_name="c", subcore_axis_name="s",
                                  num_cores=2, num_subcores=16)

    def body(cidx_ref, zeros16_ref, ones_ref, cnt_out,
             idxb0, idxb1, onesv, cntsh, semi0, semi1):
        cid = lax.axis_index("c")
        sid = lax.axis_index("s")
        wid = cid * 16 + sid
        pltpu.sync_copy(ones_ref, onesv)
        pltpu.sync_copy(zeros16_ref, cntsh.at[pl.ds(sid * SLICE, SLICE)])
        plsc.subcore_barrier()

        def scatter(idxb):
            for j in range(KGC):
                pltpu.sync_copy(onesv, cntsh.at[idxb.at[j]], add=True)

        def load_idx(g, idxb, sem):
            pltpu.async_copy(cidx_ref.at[wid, g], idxb, sem)

        def wait_idx(idxb, sem):
            pltpu.make_async_copy(cidx_ref.at[wid, 0], idxb, sem).wait()

        pltpu.sync_copy(cidx_ref.at[wid, 0], idxb0)
        load_idx(1, idxb1, semi1)

        def step(t, carry):
            last = t >= GC // 2 - 1
            scatter(idxb0)
            pl.when(~last)(lambda: load_idx(2 * t + 2, idxb0, semi0))
            wait_idx(idxb1, semi1)
            scatter(idxb1)
            pl.when(~last)(lambda: load_idx(2 * t + 3, idxb1, semi1))
            pl.when(~last)(lambda: wait_idx(idxb0, semi0))
            return carry

        lax.fori_loop(0, GC // 2, step, 0)
        plsc.subcore_barrier()
        pltpu.sync_copy(cntsh.at[pl.ds(sid * SLICE, SLICE)],
                        cnt_out.at[cid, pl.ds(sid * SLICE, SLICE)])

    run = pl.kernel(
        body,
        out_type=jax.ShapeDtypeStruct((2, ACC_ROWS, 16), jnp.float32),
        mesh=mesh,
        scratch_types=(
            pltpu.VMEM((KGC, 128), jnp.int32),
            pltpu.VMEM((KGC, 128), jnp.int32),
            pltpu.VMEM((128, 16), jnp.float32),
            pltpu.VMEM_SHARED((ACC_ROWS, 16), jnp.float32),
            pltpu.SemaphoreType.DMA,
            pltpu.SemaphoreType.DMA,
        ),
        compiler_params=pltpu.CompilerParams(use_tc_tiling_on_sc=False),
    )
    return run(cidx, zeros16, ones)


def _tc_dense(xp, accf, cnt, wstack, gs, bs, relu):
    def body(x_ref, acc_ref, cnt_ref, w_ref, gs_ref, bs_ref, o_ref):
        inv = 1.0 / jnp.maximum(cnt_ref[...], 1.0)
        invb = lax.broadcast_in_dim(inv, (R, BLK, D), (0, 1))
        scaled = acc_ref[...] * invb
        parts = [x_ref[...]] + [scaled[r] for r in range(R)]
        inp = jnp.concatenate(parts, axis=1)
        out = jnp.dot(inp, w_ref[...], preferred_element_type=jnp.float32)
        out = out * gs_ref[...] + bs_ref[...]
        if relu:
            out = jnp.maximum(out, 0.0)
        o_ref[...] = out

    grid = (NP // BLK,)
    return pl.pallas_call(
        body,
        grid=grid,
        in_specs=[
            pl.BlockSpec((BLK, D), lambda i: (i, 0)),
            pl.BlockSpec((R, BLK, D), lambda i: (0, i, 0)),
            pl.BlockSpec((R, BLK), lambda i: (0, i)),
            pl.BlockSpec((5 * D, D), lambda i: (0, 0)),
            pl.BlockSpec((1, D), lambda i: (0, 0)),
            pl.BlockSpec((1, D), lambda i: (0, 0)),
        ],
        out_specs=pl.BlockSpec((BLK, D), lambda i: (i, 0)),
        out_shape=jax.ShapeDtypeStruct((NP, D), jnp.float32),
    )(xp, accf, cnt, wstack, gs, bs)


def kernel(x, edge_index, edge_type, W0, root0, b0, bn_gamma, bn_beta,
           bn_mean, bn_var, W1, root1, b1):
    f32 = jnp.float32
    src = edge_index[0].astype(jnp.int32)
    dst = edge_index[1].astype(jnp.int32)
    et = edge_type.astype(jnp.int32)

    base = jnp.pad(src * 4, (0, EP - E))
    g4 = (base[None, :] + jnp.arange(NCHUNK, dtype=jnp.int32)[:, None]
          ).reshape(NCHUNK, 16 * G, KG, 128)
    sflat = jnp.pad(et * NP + dst, (0, EP - E), constant_values=R * NP)
    s4 = jnp.broadcast_to(sflat.reshape(1, 16 * G, KG, 128),
                          (NCHUNK, 16 * G, KG, 128))
    pidx = jnp.concatenate([g4, s4], axis=2)
    cidx = sflat.reshape(32, GC, KGC, 128)

    zeros = jnp.zeros((SLICE, CW), f32)
    zeros16 = jnp.zeros((SLICE, 16), f32)
    ones = jnp.ones((128, 16), f32)

    xp = jnp.pad(x, ((0, NP - N), (0, 0)))

    gp = bn_gamma / jnp.sqrt(bn_var + BN_EPS)
    gs0 = gp.reshape(1, D)
    bs0 = ((b0 - bn_mean) * gp + bn_beta).reshape(1, D)
    gs1 = jnp.ones((1, D), f32)
    bs1 = b1.reshape(1, D)
    wstack0 = jnp.concatenate([root0] + [W0[r] for r in range(R)], axis=0)
    wstack1 = jnp.concatenate([root1] + [W1[r] for r in range(R)], axis=0)

    cnt2 = _sc_count_pass(cidx, zeros16, ones)
    cnt = (cnt2[0, :R * NP, 0] + cnt2[1, :R * NP, 0]).reshape(R, NP)
    acc0 = _sc_edge_pass(xp.reshape(NCHUNK * NP, CW), pidx, zeros)
    h = _tc_dense(xp, acc0[:R * NP].reshape(R, NP, D), cnt,
                  wstack0, gs0, bs0, relu=True)
    acc1 = _sc_edge_pass(h.reshape(NCHUNK * NP, CW), pidx, zeros)
    out = _tc_dense(h, acc1[:R * NP].reshape(R, NP, D), cnt,
                    wstack1, gs1, bs1, relu=False)
    return out[:N]


# speedup vs baseline: 8.8432x; 1.0129x over previous
import functools

import jax
import jax.numpy as jnp
from jax import lax
from jax.experimental import pallas as pl
from jax.experimental.pallas import tpu as pltpu
from jax.experimental.pallas import tpu_sc as plsc

N = 10000
NP = 10240
E = 320000
R = 4
D = 128
CW = 32
NCHUNK = 4
KG = 4
G = 40
KGC = 8
GC = 10
T = G // 2
EP = 16 * G * KG * 128
RPS = KG * G
ACC_ROWS = R * NP + 16
SLICE = ACC_ROWS // 16
BN_EPS = 1e-5
BLK = 512


def _sc_edge_pass(table, pidx, zeros):
    mesh = plsc.VectorSubcoreMesh(core_axis_name="c", subcore_axis_name="s",
                                  num_cores=2, num_subcores=16)

    def body(table_ref, pidx_ref, zeros_ref,
             acc_out, idxb0, idxb1, rowb0, rowb1,
             accsh, semg0, semg1, semi0, semi1, sems0, sems1):
        cid = lax.axis_index("c")
        sid = lax.axis_index("s")
        for ci in range(2):
            chunk = cid * 2 + ci
            pltpu.sync_copy(zeros_ref, accsh.at[pl.ds(sid * SLICE, SLICE)])
            plsc.subcore_barrier()

            gbase = sid * G

            def fire_gathers(idxb, rowb, sem):
                for j in range(KG):
                    pltpu.async_copy(table_ref.at[idxb.at[j]],
                                     rowb.at[pl.ds(j * 128, 128)], sem)

            def drain(rowb, sem):
                pltpu.make_async_copy(table_ref.at[pl.ds(0, KG * 128)],
                                      rowb, sem).wait()

            def load_idx(g, idxb, sem):
                pltpu.async_copy(pidx_ref.at[chunk, gbase + g], idxb, sem)

            def wait_idx(idxb, sem):
                pltpu.make_async_copy(pidx_ref.at[chunk, gbase], idxb,
                                      sem).wait()

            def fire_scatters(idxb, rowb, sem):
                for j in range(KG):
                    pltpu.async_copy(rowb.at[pl.ds(j * 128, 128)],
                                     accsh.at[idxb.at[KG + j]], sem,
                                     add=True)

            pltpu.sync_copy(pidx_ref.at[chunk, gbase], idxb0)
            fire_gathers(idxb0, rowb0, semg0)
            load_idx(1, idxb1, semi1)

            def step(t, carry):
                nlast = t < T - 1
                wait_idx(idxb1, semi1)
                fire_gathers(idxb1, rowb1, semg1)
                drain(rowb0, semg0)
                fire_scatters(idxb0, rowb0, sems0)
                drain(rowb0, sems0)
                pl.when(nlast)(lambda: load_idx(2 * t + 2, idxb0, semi0))
                drain(rowb1, semg1)
                fire_scatters(idxb1, rowb1, sems1)

                def refill_a():
                    wait_idx(idxb0, semi0)
                    fire_gathers(idxb0, rowb0, semg0)
                pl.when(nlast)(refill_a)
                drain(rowb1, sems1)
                pl.when(nlast)(lambda: load_idx(2 * t + 3, idxb1, semi1))
                return carry

            lax.fori_loop(0, T, step, 0)
            plsc.subcore_barrier()
            pltpu.sync_copy(accsh.at[pl.ds(sid * SLICE, SLICE)],
                            acc_out.at[pl.ds(sid * SLICE, SLICE),
                                       pl.ds(chunk * CW, CW)])

    run = pl.kernel(
        body,
        out_type=jax.ShapeDtypeStruct((ACC_ROWS, D), jnp.float32),
        mesh=mesh,
        scratch_types=(
            pltpu.VMEM((2 * KG, 128), jnp.int32),
            pltpu.VMEM((2 * KG, 128), jnp.int32),
            pltpu.VMEM((KG * 128, CW), jnp.float32),
            pltpu.VMEM((KG * 128, CW), jnp.float32),
            pltpu.VMEM_SHARED((ACC_ROWS, CW), jnp.float32),
            pltpu.SemaphoreType.DMA,
            pltpu.SemaphoreType.DMA,
            pltpu.SemaphoreType.DMA,
            pltpu.SemaphoreType.DMA,
            pltpu.SemaphoreType.DMA,
            pltpu.SemaphoreType.DMA,
        ),
        compiler_params=pltpu.CompilerParams(use_tc_tiling_on_sc=False),
    )
    return run(table, pidx, zeros)


def _sc_count_pass(cidx, zeros16, ones):
    mesh = plsc.VectorSubcoreMesh(core_axis_name="c", subcore_axis_name="s",
                                  num_cores=2, num_subcores=16)

    def body(cidx_ref, zeros16_ref, ones_ref, cnt_out,
             idxb0, idxb1, onesv, cntsh, semi0, semi1):
        cid = lax.axis_index("c")
        sid = lax.axis_index("s")
        wid = cid * 16 + sid
        pltpu.sync_copy(ones_ref, onesv)
        pltpu.sync_copy(zeros16_ref, cntsh.at[pl.ds(sid * SLICE, SLICE)])
        plsc.subcore_barrier()

        def scatter(idxb):
            for j in range(KGC):
                pltpu.sync_copy(onesv, cntsh.at[idxb.at[j]], add=True)

        def load_idx(g, idxb, sem):
            pltpu.async_copy(cidx_ref.at[wid, g], idxb, sem)

        def wait_idx(idxb, sem):
            pltpu.make_async_copy(cidx_ref.at[wid, 0], idxb, sem).wait()

        pltpu.sync_copy(cidx_ref.at[wid, 0], idxb0)
        load_idx(1, idxb1, semi1)

        def step(t, carry):
            last = t >= GC // 2 - 1
            scatter(idxb0)
            pl.when(~last)(lambda: load_idx(2 * t + 2, idxb0, semi0))
            wait_idx(idxb1, semi1)
            scatter(idxb1)
            pl.when(~last)(lambda: load_idx(2 * t + 3, idxb1, semi1))
            pl.when(~last)(lambda: wait_idx(idxb0, semi0))
            return carry

        lax.fori_loop(0, GC // 2, step, 0)
        plsc.subcore_barrier()
        pltpu.sync_copy(cntsh.at[pl.ds(sid * SLICE, SLICE)],
                        cnt_out.at[cid, pl.ds(sid * SLICE, SLICE)])

    run = pl.kernel(
        body,
        out_type=jax.ShapeDtypeStruct((2, ACC_ROWS, 16), jnp.float32),
        mesh=mesh,
        scratch_types=(
            pltpu.VMEM((KGC, 128), jnp.int32),
            pltpu.VMEM((KGC, 128), jnp.int32),
            pltpu.VMEM((128, 16), jnp.float32),
            pltpu.VMEM_SHARED((ACC_ROWS, 16), jnp.float32),
            pltpu.SemaphoreType.DMA,
            pltpu.SemaphoreType.DMA,
        ),
        compiler_params=pltpu.CompilerParams(use_tc_tiling_on_sc=False),
    )
    return run(cidx, zeros16, ones)


def _tc_dense(xp, accf, cnt, wstack, gs, bs, relu):
    def body(x_ref, acc_ref, cnt_ref, w_ref, gs_ref, bs_ref, o_ref):
        inv = 1.0 / jnp.maximum(cnt_ref[...], 1.0)
        invb = lax.broadcast_in_dim(inv, (R, BLK, D), (0, 1))
        scaled = acc_ref[...] * invb
        parts = [x_ref[...]] + [scaled[r] for r in range(R)]
        inp = jnp.concatenate(parts, axis=1)
        out = jnp.dot(inp, w_ref[...], preferred_element_type=jnp.float32)
        out = out * gs_ref[...] + bs_ref[...]
        if relu:
            out = jnp.maximum(out, 0.0)
        o_ref[...] = out

    grid = (NP // BLK,)
    return pl.pallas_call(
        body,
        grid=grid,
        in_specs=[
            pl.BlockSpec((BLK, D), lambda i: (i, 0)),
            pl.BlockSpec((R, BLK, D), lambda i: (0, i, 0)),
            pl.BlockSpec((R, BLK), lambda i: (0, i)),
            pl.BlockSpec((5 * D, D), lambda i: (0, 0)),
            pl.BlockSpec((1, D), lambda i: (0, 0)),
            pl.BlockSpec((1, D), lambda i: (0, 0)),
        ],
        out_specs=pl.BlockSpec((BLK, D), lambda i: (i, 0)),
        out_shape=jax.ShapeDtypeStruct((NP, D), jnp.float32),
    )(xp, accf, cnt, wstack, gs, bs)


def kernel(x, edge_index, edge_type, W0, root0, b0, bn_gamma, bn_beta,
           bn_mean, bn_var, W1, root1, b1):
    f32 = jnp.float32
    src = edge_index[0].astype(jnp.int32)
    dst = edge_index[1].astype(jnp.int32)
    et = edge_type.astype(jnp.int32)

    base = jnp.pad(src * 4, (0, EP - E))
    g4 = (base[None, :] + jnp.arange(NCHUNK, dtype=jnp.int32)[:, None]
          ).reshape(NCHUNK, 16 * G, KG, 128)
    sflat = jnp.pad(et * NP + dst, (0, EP - E), constant_values=R * NP)
    s4 = jnp.broadcast_to(sflat.reshape(1, 16 * G, KG, 128),
                          (NCHUNK, 16 * G, KG, 128))
    pidx = jnp.concatenate([g4, s4], axis=2)
    cidx = sflat.reshape(32, GC, KGC, 128)

    zeros = jnp.zeros((SLICE, CW), f32)
    zeros16 = jnp.zeros((SLICE, 16), f32)
    ones = jnp.ones((128, 16), f32)

    xp = jnp.pad(x, ((0, NP - N), (0, 0)))

    gp = bn_gamma / jnp.sqrt(bn_var + BN_EPS)
    gs0 = gp.reshape(1, D)
    bs0 = ((b0 - bn_mean) * gp + bn_beta).reshape(1, D)
    gs1 = jnp.ones((1, D), f32)
    bs1 = b1.reshape(1, D)
    wstack0 = jnp.concatenate([root0] + [W0[r] for r in range(R)], axis=0)
    wstack1 = jnp.concatenate([root1] + [W1[r] for r in range(R)], axis=0)

    cnt2 = _sc_count_pass(cidx, zeros16, ones)
    cnt = (cnt2[0, :R * NP, 0] + cnt2[1, :R * NP, 0]).reshape(R, NP)
    acc0 = _sc_edge_pass(xp.reshape(NCHUNK * NP, CW), pidx, zeros)
    h = _tc_dense(xp, acc0[:R * NP].reshape(R, NP, D), cnt,
                  wstack0, gs0, bs0, relu=True)
    acc1 = _sc_edge_pass(h.reshape(NCHUNK * NP, CW), pidx, zeros)
    out = _tc_dense(h, acc1[:R * NP].reshape(R, NP, D), cnt,
                    wstack1, gs1, bs1, relu=False)
    return out[:N]


# speedup vs baseline: 15.2652x; 1.7262x over previous
import functools

import jax
import jax.numpy as jnp
from jax import lax
from jax.experimental import pallas as pl
from jax.experimental.pallas import tpu as pltpu
from jax.experimental.pallas import tpu_sc as plsc

N = 10000
NP = 10240
E = 320000
R = 4
D = 128
CW = 32
NCHUNK = 4
KG = 2
G = 80
KGC = 8
GC = 10
T = G // 2
EP = 16 * G * KG * 128
RPS = KG * G
ACC_ROWS = R * NP + 16
SLICE = ACC_ROWS // 16
BN_EPS = 1e-5
BLK = 512


def _sc_edge_pass(table, pidx, zeros):
    mesh = plsc.VectorSubcoreMesh(core_axis_name="c", subcore_axis_name="s",
                                  num_cores=2, num_subcores=16)

    def body(table_ref, pidx_ref, zeros_ref,
             acc_out, idxb0, idxb1, rowb0, rowb1,
             accsh, tablesh, semg0, semg1, semi0, semi1, sems0, sems1):
        cid = lax.axis_index("c")
        sid = lax.axis_index("s")
        trows = NP // 16
        for ci in range(2):
            chunk = cid * 2 + ci
            pltpu.sync_copy(zeros_ref, accsh.at[pl.ds(sid * SLICE, SLICE)])
            pltpu.sync_copy(table_ref.at[pl.ds(sid * trows, trows),
                                         pl.ds(chunk * CW, CW)],
                            tablesh.at[pl.ds(sid * trows, trows)])
            plsc.subcore_barrier()

            gbase = sid * G

            def fire_gathers(idxb, rowb, sem):
                for j in range(KG):
                    pltpu.async_copy(tablesh.at[idxb.at[j]],
                                     rowb.at[pl.ds(j * 128, 128)], sem)

            def drain(rowb, sem):
                pltpu.make_async_copy(table_ref.at[pl.ds(0, KG * 128),
                                                   pl.ds(0, CW)],
                                      rowb, sem).wait()

            def load_idx(g, idxb, sem):
                pltpu.async_copy(pidx_ref.at[gbase + g], idxb, sem)

            def wait_idx(idxb, sem):
                pltpu.make_async_copy(pidx_ref.at[gbase], idxb,
                                      sem).wait()

            def fire_scatters(idxb, rowb, sem):
                for j in range(KG):
                    pltpu.async_copy(rowb.at[pl.ds(j * 128, 128)],
                                     accsh.at[idxb.at[KG + j]], sem,
                                     add=True)

            pltpu.sync_copy(pidx_ref.at[gbase], idxb0)
            fire_gathers(idxb0, rowb0, semg0)
            load_idx(1, idxb1, semi1)

            def step(t, carry):
                nlast = t < T - 1
                wait_idx(idxb1, semi1)
                fire_gathers(idxb1, rowb1, semg1)
                drain(rowb0, semg0)
                fire_scatters(idxb0, rowb0, sems0)
                drain(rowb0, sems0)
                pl.when(nlast)(lambda: load_idx(2 * t + 2, idxb0, semi0))
                drain(rowb1, semg1)
                fire_scatters(idxb1, rowb1, sems1)

                def refill_a():
                    wait_idx(idxb0, semi0)
                    fire_gathers(idxb0, rowb0, semg0)
                pl.when(nlast)(refill_a)
                drain(rowb1, sems1)
                pl.when(nlast)(lambda: load_idx(2 * t + 3, idxb1, semi1))
                return carry

            lax.fori_loop(0, T, step, 0)
            plsc.subcore_barrier()
            pltpu.sync_copy(accsh.at[pl.ds(sid * SLICE, SLICE)],
                            acc_out.at[pl.ds(sid * SLICE, SLICE),
                                       pl.ds(chunk * CW, CW)])

    run = pl.kernel(
        body,
        out_type=jax.ShapeDtypeStruct((ACC_ROWS, D), jnp.float32),
        mesh=mesh,
        scratch_types=(
            pltpu.VMEM((2 * KG, 128), jnp.int32),
            pltpu.VMEM((2 * KG, 128), jnp.int32),
            pltpu.VMEM((KG * 128, CW), jnp.float32),
            pltpu.VMEM((KG * 128, CW), jnp.float32),
            pltpu.VMEM_SHARED((ACC_ROWS, CW), jnp.float32),
            pltpu.VMEM_SHARED((NP, CW), jnp.float32),
            pltpu.SemaphoreType.DMA,
            pltpu.SemaphoreType.DMA,
            pltpu.SemaphoreType.DMA,
            pltpu.SemaphoreType.DMA,
            pltpu.SemaphoreType.DMA,
            pltpu.SemaphoreType.DMA,
        ),
        compiler_params=pltpu.CompilerParams(use_tc_tiling_on_sc=False),
    )
    return run(table, pidx, zeros)


def _sc_count_pass(cidx, zeros16, ones):
    mesh = plsc.VectorSubcoreMesh(core_axis_name="c", subcore_axis_name="s",
                                  num_cores=2, num_subcores=16)

    def body(cidx_ref, zeros16_ref, ones_ref, cnt_out,
             idxb0, idxb1, onesv, cntsh, semi0, semi1):
        cid = lax.axis_index("c")
        sid = lax.axis_index("s")
        wid = cid * 16 + sid
        pltpu.sync_copy(ones_ref, onesv)
        pltpu.sync_copy(zeros16_ref, cntsh.at[pl.ds(sid * SLICE, SLICE)])
        plsc.subcore_barrier()

        def scatter(idxb):
            for j in range(KGC):
                pltpu.sync_copy(onesv, cntsh.at[idxb.at[j]], add=True)

        def load_idx(g, idxb, sem):
            pltpu.async_copy(cidx_ref.at[wid, g], idxb, sem)

        def wait_idx(idxb, sem):
            pltpu.make_async_copy(cidx_ref.at[wid, 0], idxb, sem).wait()

        pltpu.sync_copy(cidx_ref.at[wid, 0], idxb0)
        load_idx(1, idxb1, semi1)

        def step(t, carry):
            last = t >= GC // 2 - 1
            scatter(idxb0)
            pl.when(~last)(lambda: load_idx(2 * t + 2, idxb0, semi0))
            wait_idx(idxb1, semi1)
            scatter(idxb1)
            pl.when(~last)(lambda: load_idx(2 * t + 3, idxb1, semi1))
            pl.when(~last)(lambda: wait_idx(idxb0, semi0))
            return carry

        lax.fori_loop(0, GC // 2, step, 0)
        plsc.subcore_barrier()
        pltpu.sync_copy(cntsh.at[pl.ds(sid * SLICE, SLICE)],
                        cnt_out.at[cid, pl.ds(sid * SLICE, SLICE)])

    run = pl.kernel(
        body,
        out_type=jax.ShapeDtypeStruct((2, ACC_ROWS, 16), jnp.float32),
        mesh=mesh,
        scratch_types=(
            pltpu.VMEM((KGC, 128), jnp.int32),
            pltpu.VMEM((KGC, 128), jnp.int32),
            pltpu.VMEM((128, 16), jnp.float32),
            pltpu.VMEM_SHARED((ACC_ROWS, 16), jnp.float32),
            pltpu.SemaphoreType.DMA,
            pltpu.SemaphoreType.DMA,
        ),
        compiler_params=pltpu.CompilerParams(use_tc_tiling_on_sc=False),
    )
    return run(cidx, zeros16, ones)


def _tc_dense(xp, accf, cnt, wstack, gs, bs, relu):
    def body(x_ref, acc_ref, cnt_ref, w_ref, gs_ref, bs_ref, o_ref):
        inv = 1.0 / jnp.maximum(cnt_ref[...], 1.0)
        invb = lax.broadcast_in_dim(inv, (R, BLK, D), (0, 1))
        scaled = acc_ref[...] * invb
        parts = [x_ref[...]] + [scaled[r] for r in range(R)]
        inp = jnp.concatenate(parts, axis=1)
        out = jnp.dot(inp, w_ref[...], preferred_element_type=jnp.float32)
        out = out * gs_ref[...] + bs_ref[...]
        if relu:
            out = jnp.maximum(out, 0.0)
        o_ref[...] = out

    grid = (NP // BLK,)
    return pl.pallas_call(
        body,
        grid=grid,
        in_specs=[
            pl.BlockSpec((BLK, D), lambda i: (i, 0)),
            pl.BlockSpec((R, BLK, D), lambda i: (0, i, 0)),
            pl.BlockSpec((R, BLK), lambda i: (0, i)),
            pl.BlockSpec((5 * D, D), lambda i: (0, 0)),
            pl.BlockSpec((1, D), lambda i: (0, 0)),
            pl.BlockSpec((1, D), lambda i: (0, 0)),
        ],
        out_specs=pl.BlockSpec((BLK, D), lambda i: (i, 0)),
        out_shape=jax.ShapeDtypeStruct((NP, D), jnp.float32),
    )(xp, accf, cnt, wstack, gs, bs)


def kernel(x, edge_index, edge_type, W0, root0, b0, bn_gamma, bn_beta,
           bn_mean, bn_var, W1, root1, b1):
    f32 = jnp.float32
    src = edge_index[0].astype(jnp.int32)
    dst = edge_index[1].astype(jnp.int32)
    et = edge_type.astype(jnp.int32)

    ar = jnp.arange(EP - E, dtype=jnp.int32)
    gflat = jnp.concatenate([src, ar % N]).reshape(16 * G, KG, 128)
    sflat = jnp.concatenate([et * NP + dst, R * NP + (ar % 16)])
    pidx = jnp.concatenate([gflat, sflat.reshape(16 * G, KG, 128)], axis=1)
    cidx = sflat.reshape(32, GC, KGC, 128)

    zeros = jnp.zeros((SLICE, CW), f32)
    zeros16 = jnp.zeros((SLICE, 16), f32)
    ones = jnp.ones((128, 16), f32)

    xp = jnp.pad(x, ((0, NP - N), (0, 0)))

    gp = bn_gamma / jnp.sqrt(bn_var + BN_EPS)
    gs0 = gp.reshape(1, D)
    bs0 = ((b0 - bn_mean) * gp + bn_beta).reshape(1, D)
    gs1 = jnp.ones((1, D), f32)
    bs1 = b1.reshape(1, D)
    wstack0 = jnp.concatenate([root0] + [W0[r] for r in range(R)], axis=0)
    wstack1 = jnp.concatenate([root1] + [W1[r] for r in range(R)], axis=0)

    cnt2 = _sc_count_pass(cidx, zeros16, ones)
    cnt = (cnt2[0, :R * NP, 0] + cnt2[1, :R * NP, 0]).reshape(R, NP)
    acc0 = _sc_edge_pass(xp, pidx, zeros)
    h = _tc_dense(xp, acc0[:R * NP].reshape(R, NP, D), cnt,
                  wstack0, gs0, bs0, relu=True)
    acc1 = _sc_edge_pass(h, pidx, zeros)
    out = _tc_dense(h, acc1[:R * NP].reshape(R, NP, D), cnt,
                    wstack1, gs1, bs1, relu=False)
    return out[:N]


# speedup vs baseline: 15.3295x; 1.0042x over previous
import functools

import jax
import jax.numpy as jnp
from jax import lax
from jax.experimental import pallas as pl
from jax.experimental.pallas import tpu as pltpu
from jax.experimental.pallas import tpu_sc as plsc

N = 10000
NP = 10240
E = 320000
R = 4
D = 128
CW = 32
NCHUNK = 4
KG = 2
G = 80
KGC = 8
GC = 10
T = G // 2
EP = 16 * G * KG * 128
RPS = KG * G
ACC_ROWS = R * NP + 16
SLICE = ACC_ROWS // 16
BN_EPS = 1e-5
BLK = 512


def _sc_edge_pass(table, pidx, zeros):
    mesh = plsc.VectorSubcoreMesh(core_axis_name="c", subcore_axis_name="s",
                                  num_cores=2, num_subcores=16)

    def body(table_ref, pidx_ref, zeros_ref,
             acc_out, idxb0, idxb1, rowb0, rowb1,
             accsh, tablesh, semg0, semg1, semi0, semi1, sems0, sems1):
        cid = lax.axis_index("c")
        sid = lax.axis_index("s")
        trows = NP // 16
        for ci in range(2):
            chunk = cid * 2 + ci
            dz = pltpu.async_copy(zeros_ref,
                                  accsh.at[pl.ds(sid * SLICE, SLICE)], semi0)
            ds_ = pltpu.async_copy(table_ref.at[pl.ds(sid * trows, trows),
                                                pl.ds(chunk * CW, CW)],
                                   tablesh.at[pl.ds(sid * trows, trows)],
                                   semi1)
            dz.wait()
            ds_.wait()
            plsc.subcore_barrier()

            gbase = sid * G

            def fire_gathers(idxb, rowb, sem):
                for j in range(KG):
                    pltpu.async_copy(tablesh.at[idxb.at[j]],
                                     rowb.at[pl.ds(j * 128, 128)], sem)

            def drain(rowb, sem):
                pltpu.make_async_copy(table_ref.at[pl.ds(0, KG * 128),
                                                   pl.ds(0, CW)],
                                      rowb, sem).wait()

            def load_idx(g, idxb, sem):
                pltpu.async_copy(pidx_ref.at[gbase + g], idxb, sem)

            def wait_idx(idxb, sem):
                pltpu.make_async_copy(pidx_ref.at[gbase], idxb,
                                      sem).wait()

            def fire_scatters(idxb, rowb, sem):
                for j in range(KG):
                    pltpu.async_copy(rowb.at[pl.ds(j * 128, 128)],
                                     accsh.at[idxb.at[KG + j]], sem,
                                     add=True)

            pltpu.sync_copy(pidx_ref.at[gbase], idxb0)
            fire_gathers(idxb0, rowb0, semg0)
            load_idx(1, idxb1, semi1)

            def step(t, carry):
                nlast = t < T - 1
                wait_idx(idxb1, semi1)
                fire_gathers(idxb1, rowb1, semg1)
                drain(rowb0, semg0)
                fire_scatters(idxb0, rowb0, sems0)
                drain(rowb0, sems0)
                pl.when(nlast)(lambda: load_idx(2 * t + 2, idxb0, semi0))
                drain(rowb1, semg1)
                fire_scatters(idxb1, rowb1, sems1)

                def refill_a():
                    wait_idx(idxb0, semi0)
                    fire_gathers(idxb0, rowb0, semg0)
                pl.when(nlast)(refill_a)
                drain(rowb1, sems1)
                pl.when(nlast)(lambda: load_idx(2 * t + 3, idxb1, semi1))
                return carry

            lax.fori_loop(0, T, step, 0)
            plsc.subcore_barrier()
            pltpu.sync_copy(accsh.at[pl.ds(sid * SLICE, SLICE)],
                            acc_out.at[pl.ds(sid * SLICE, SLICE),
                                       pl.ds(chunk * CW, CW)])

    run = pl.kernel(
        body,
        out_type=jax.ShapeDtypeStruct((ACC_ROWS, D), jnp.float32),
        mesh=mesh,
        scratch_types=(
            pltpu.VMEM((2 * KG, 128), jnp.int32),
            pltpu.VMEM((2 * KG, 128), jnp.int32),
            pltpu.VMEM((KG * 128, CW), jnp.float32),
            pltpu.VMEM((KG * 128, CW), jnp.float32),
            pltpu.VMEM_SHARED((ACC_ROWS, CW), jnp.float32),
            pltpu.VMEM_SHARED((NP, CW), jnp.float32),
            pltpu.SemaphoreType.DMA,
            pltpu.SemaphoreType.DMA,
            pltpu.SemaphoreType.DMA,
            pltpu.SemaphoreType.DMA,
            pltpu.SemaphoreType.DMA,
            pltpu.SemaphoreType.DMA,
        ),
        compiler_params=pltpu.CompilerParams(use_tc_tiling_on_sc=False),
    )
    return run(table, pidx, zeros)


def _sc_count_pass(cidx, zeros16, ones):
    mesh = plsc.VectorSubcoreMesh(core_axis_name="c", subcore_axis_name="s",
                                  num_cores=2, num_subcores=16)

    def body(cidx_ref, zeros16_ref, ones_ref, cnt_out,
             idxb0, idxb1, onesv, cntsh, semi0, semi1):
        cid = lax.axis_index("c")
        sid = lax.axis_index("s")
        wid = cid * 16 + sid
        pltpu.sync_copy(ones_ref, onesv)
        pltpu.sync_copy(zeros16_ref, cntsh.at[pl.ds(sid * SLICE, SLICE)])
        plsc.subcore_barrier()

        def scatter(idxb):
            for j in range(KGC):
                pltpu.sync_copy(onesv, cntsh.at[idxb.at[j]], add=True)

        def load_idx(g, idxb, sem):
            pltpu.async_copy(cidx_ref.at[wid, g], idxb, sem)

        def wait_idx(idxb, sem):
            pltpu.make_async_copy(cidx_ref.at[wid, 0], idxb, sem).wait()

        pltpu.sync_copy(cidx_ref.at[wid, 0], idxb0)
        load_idx(1, idxb1, semi1)

        def step(t, carry):
            last = t >= GC // 2 - 1
            scatter(idxb0)
            pl.when(~last)(lambda: load_idx(2 * t + 2, idxb0, semi0))
            wait_idx(idxb1, semi1)
            scatter(idxb1)
            pl.when(~last)(lambda: load_idx(2 * t + 3, idxb1, semi1))
            pl.when(~last)(lambda: wait_idx(idxb0, semi0))
            return carry

        lax.fori_loop(0, GC // 2, step, 0)
        plsc.subcore_barrier()
        pltpu.sync_copy(cntsh.at[pl.ds(sid * SLICE, SLICE)],
                        cnt_out.at[cid, pl.ds(sid * SLICE, SLICE)])

    run = pl.kernel(
        body,
        out_type=jax.ShapeDtypeStruct((2, ACC_ROWS, 16), jnp.float32),
        mesh=mesh,
        scratch_types=(
            pltpu.VMEM((KGC, 128), jnp.int32),
            pltpu.VMEM((KGC, 128), jnp.int32),
            pltpu.VMEM((128, 16), jnp.float32),
            pltpu.VMEM_SHARED((ACC_ROWS, 16), jnp.float32),
            pltpu.SemaphoreType.DMA,
            pltpu.SemaphoreType.DMA,
        ),
        compiler_params=pltpu.CompilerParams(use_tc_tiling_on_sc=False),
    )
    return run(cidx, zeros16, ones)


def _tc_dense(xp, accf, cnt, wstack, gs, bs, relu):
    def body(x_ref, acc_ref, cnt_ref, w_ref, gs_ref, bs_ref, o_ref):
        inv = 1.0 / jnp.maximum(cnt_ref[...], 1.0)
        invb = lax.broadcast_in_dim(inv, (R, BLK, D), (0, 1))
        scaled = acc_ref[...] * invb
        parts = [x_ref[...]] + [scaled[r] for r in range(R)]
        inp = jnp.concatenate(parts, axis=1)
        out = jnp.dot(inp, w_ref[...], preferred_element_type=jnp.float32)
        out = out * gs_ref[...] + bs_ref[...]
        if relu:
            out = jnp.maximum(out, 0.0)
        o_ref[...] = out

    grid = (NP // BLK,)
    return pl.pallas_call(
        body,
        grid=grid,
        in_specs=[
            pl.BlockSpec((BLK, D), lambda i: (i, 0)),
            pl.BlockSpec((R, BLK, D), lambda i: (0, i, 0)),
            pl.BlockSpec((R, BLK), lambda i: (0, i)),
            pl.BlockSpec((5 * D, D), lambda i: (0, 0)),
            pl.BlockSpec((1, D), lambda i: (0, 0)),
            pl.BlockSpec((1, D), lambda i: (0, 0)),
        ],
        out_specs=pl.BlockSpec((BLK, D), lambda i: (i, 0)),
        out_shape=jax.ShapeDtypeStruct((NP, D), jnp.float32),
    )(xp, accf, cnt, wstack, gs, bs)


def kernel(x, edge_index, edge_type, W0, root0, b0, bn_gamma, bn_beta,
           bn_mean, bn_var, W1, root1, b1):
    f32 = jnp.float32
    src = edge_index[0].astype(jnp.int32)
    dst = edge_index[1].astype(jnp.int32)
    et = edge_type.astype(jnp.int32)

    ar = jnp.arange(EP - E, dtype=jnp.int32)
    gflat = jnp.concatenate([src, ar % N]).reshape(16 * G, KG, 128)
    sflat = jnp.concatenate([et * NP + dst, R * NP + (ar % 16)])
    pidx = jnp.concatenate([gflat, sflat.reshape(16 * G, KG, 128)], axis=1)
    cidx = sflat.reshape(32, GC, KGC, 128)

    zeros = jnp.zeros((SLICE, CW), f32)
    zeros16 = jnp.zeros((SLICE, 16), f32)
    ones = jnp.ones((128, 16), f32)

    xp = jnp.pad(x, ((0, NP - N), (0, 0)))

    gp = bn_gamma / jnp.sqrt(bn_var + BN_EPS)
    gs0 = gp.reshape(1, D)
    bs0 = ((b0 - bn_mean) * gp + bn_beta).reshape(1, D)
    gs1 = jnp.ones((1, D), f32)
    bs1 = b1.reshape(1, D)
    wstack0 = jnp.concatenate([root0] + [W0[r] for r in range(R)], axis=0)
    wstack1 = jnp.concatenate([root1] + [W1[r] for r in range(R)], axis=0)

    cnt2 = _sc_count_pass(cidx, zeros16, ones)
    cnt = (cnt2[0, :R * NP, 0] + cnt2[1, :R * NP, 0]).reshape(R, NP)
    acc0 = _sc_edge_pass(xp, pidx, zeros)
    h = _tc_dense(xp, acc0[:R * NP].reshape(R, NP, D), cnt,
                  wstack0, gs0, bs0, relu=True)
    acc1 = _sc_edge_pass(h, pidx, zeros)
    out = _tc_dense(h, acc1[:R * NP].reshape(R, NP, D), cnt,
                    wstack1, gs1, bs1, relu=False)
    return out[:N]


# speedup vs baseline: 15.3303x; 1.0001x over previous
import jax
import jax.numpy as jnp
from jax import lax
from jax.experimental import pallas as pl
from jax.experimental.pallas import tpu as pltpu
from jax.experimental.pallas import tpu_sc as plsc

N = 10000
NP = 10240
E = 320000
R = 4
D = 128
CW = 32
KG = 2
G = 80
KGC = 8
GC = 10
T = G // 2
EP = 16 * G * KG * 128
ACC_ROWS = R * NP + 16
SLICE = ACC_ROWS // 16
BN_EPS = 1e-5
BLK = 512


def _sc_edge_pass(table, pidx, zeros):
    mesh = plsc.VectorSubcoreMesh(core_axis_name="c", subcore_axis_name="s",
                                  num_cores=2, num_subcores=16)

    def body(table_ref, pidx_ref, zeros_ref,
             acc_out, idxb0, idxb1, rowb0, rowb1,
             accsh, tablesh, semg0, semg1, semi0, semi1, sems0, sems1):
        cid = lax.axis_index("c")
        sid = lax.axis_index("s")
        trows = NP // 16
        for ci in range(2):
            chunk = cid * 2 + ci
            dz = pltpu.async_copy(zeros_ref,
                                  accsh.at[pl.ds(sid * SLICE, SLICE)], semi0)
            ds_ = pltpu.async_copy(table_ref.at[pl.ds(sid * trows, trows),
                                                pl.ds(chunk * CW, CW)],
                                   tablesh.at[pl.ds(sid * trows, trows)],
                                   semi1)
            dz.wait()
            ds_.wait()
            plsc.subcore_barrier()

            gbase = sid * G

            def fire_gathers(idxb, rowb, sem):
                for j in range(KG):
                    pltpu.async_copy(tablesh.at[idxb.at[j]],
                                     rowb.at[pl.ds(j * 128, 128)], sem)

            def drain(rowb, sem):
                pltpu.make_async_copy(table_ref.at[pl.ds(0, KG * 128),
                                                   pl.ds(0, CW)],
                                      rowb, sem).wait()

            def load_idx(g, idxb, sem):
                pltpu.async_copy(pidx_ref.at[gbase + g], idxb, sem)

            def wait_idx(idxb, sem):
                pltpu.make_async_copy(pidx_ref.at[gbase], idxb,
                                      sem).wait()

            def fire_scatters(idxb, rowb, sem):
                for j in range(KG):
                    pltpu.async_copy(rowb.at[pl.ds(j * 128, 128)],
                                     accsh.at[idxb.at[KG + j]], sem,
                                     add=True)

            pltpu.sync_copy(pidx_ref.at[gbase], idxb0)
            fire_gathers(idxb0, rowb0, semg0)
            load_idx(1, idxb1, semi1)

            def step(t, carry):
                nlast = t < T - 1
                wait_idx(idxb1, semi1)
                fire_gathers(idxb1, rowb1, semg1)
                drain(rowb0, semg0)
                fire_scatters(idxb0, rowb0, sems0)
                drain(rowb0, sems0)
                pl.when(nlast)(lambda: load_idx(2 * t + 2, idxb0, semi0))
                drain(rowb1, semg1)
                fire_scatters(idxb1, rowb1, sems1)

                def refill_a():
                    wait_idx(idxb0, semi0)
                    fire_gathers(idxb0, rowb0, semg0)
                pl.when(nlast)(refill_a)
                drain(rowb1, sems1)
                pl.when(nlast)(lambda: load_idx(2 * t + 3, idxb1, semi1))
                return carry

            lax.fori_loop(0, T, step, 0)
            plsc.subcore_barrier()
            pltpu.sync_copy(accsh.at[pl.ds(sid * SLICE, SLICE)],
                            acc_out.at[pl.ds(sid * SLICE, SLICE),
                                       pl.ds(chunk * CW, CW)])

    run = pl.kernel(
        body,
        out_type=jax.ShapeDtypeStruct((ACC_ROWS, D), jnp.float32),
        mesh=mesh,
        scratch_types=(
            pltpu.VMEM((2 * KG, 128), jnp.int32),
            pltpu.VMEM((2 * KG, 128), jnp.int32),
            pltpu.VMEM((KG * 128, CW), jnp.float32),
            pltpu.VMEM((KG * 128, CW), jnp.float32),
            pltpu.VMEM_SHARED((ACC_ROWS, CW), jnp.float32),
            pltpu.VMEM_SHARED((NP, CW), jnp.float32),
            pltpu.SemaphoreType.DMA,
            pltpu.SemaphoreType.DMA,
            pltpu.SemaphoreType.DMA,
            pltpu.SemaphoreType.DMA,
            pltpu.SemaphoreType.DMA,
            pltpu.SemaphoreType.DMA,
        ),
        compiler_params=pltpu.CompilerParams(use_tc_tiling_on_sc=False),
    )
    return run(table, pidx, zeros)


def _sc_count_pass(cidx, zeros16, ones):
    mesh = plsc.VectorSubcoreMesh(core_axis_name="c", subcore_axis_name="s",
                                  num_cores=2, num_subcores=16)

    def body(cidx_ref, zeros16_ref, ones_ref, cnt_out,
             idxb0, idxb1, onesv, cntsh, semi0, semi1):
        cid = lax.axis_index("c")
        sid = lax.axis_index("s")
        wid = cid * 16 + sid
        pltpu.sync_copy(ones_ref, onesv)
        pltpu.sync_copy(zeros16_ref, cntsh.at[pl.ds(sid * SLICE, SLICE)])
        plsc.subcore_barrier()

        def scatter(idxb):
            for j in range(KGC):
                pltpu.sync_copy(onesv, cntsh.at[idxb.at[j]], add=True)

        def load_idx(g, idxb, sem):
            pltpu.async_copy(cidx_ref.at[wid, g], idxb, sem)

        def wait_idx(idxb, sem):
            pltpu.make_async_copy(cidx_ref.at[wid, 0], idxb, sem).wait()

        pltpu.sync_copy(cidx_ref.at[wid, 0], idxb0)
        load_idx(1, idxb1, semi1)

        def step(t, carry):
            last = t >= GC // 2 - 1
            scatter(idxb0)
            pl.when(~last)(lambda: load_idx(2 * t + 2, idxb0, semi0))
            wait_idx(idxb1, semi1)
            scatter(idxb1)
            pl.when(~last)(lambda: load_idx(2 * t + 3, idxb1, semi1))
            pl.when(~last)(lambda: wait_idx(idxb0, semi0))
            return carry

        lax.fori_loop(0, GC // 2, step, 0)
        plsc.subcore_barrier()
        pltpu.sync_copy(cntsh.at[pl.ds(sid * SLICE, SLICE)],
                        cnt_out.at[cid, pl.ds(sid * SLICE, SLICE)])

    run = pl.kernel(
        body,
        out_type=jax.ShapeDtypeStruct((2, ACC_ROWS, 16), jnp.float32),
        mesh=mesh,
        scratch_types=(
            pltpu.VMEM((KGC, 128), jnp.int32),
            pltpu.VMEM((KGC, 128), jnp.int32),
            pltpu.VMEM((128, 16), jnp.float32),
            pltpu.VMEM_SHARED((ACC_ROWS, 16), jnp.float32),
            pltpu.SemaphoreType.DMA,
            pltpu.SemaphoreType.DMA,
        ),
        compiler_params=pltpu.CompilerParams(use_tc_tiling_on_sc=False),
    )
    return run(cidx, zeros16, ones)


def _tc_dense(xp, accf, cnt, wstack, gs, bs, relu):
    def body(x_ref, acc_ref, cnt_ref, w_ref, gs_ref, bs_ref, o_ref):
        inv = 1.0 / jnp.maximum(cnt_ref[...], 1.0)
        invb = lax.broadcast_in_dim(inv, (R, BLK, D), (0, 1))
        scaled = acc_ref[...] * invb
        parts = [x_ref[...]] + [scaled[r] for r in range(R)]
        inp = jnp.concatenate(parts, axis=1)
        out = jnp.dot(inp, w_ref[...], preferred_element_type=jnp.float32)
        out = out * gs_ref[...] + bs_ref[...]
        if relu:
            out = jnp.maximum(out, 0.0)
        o_ref[...] = out

    grid = (NP // BLK,)
    return pl.pallas_call(
        body,
        grid=grid,
        in_specs=[
            pl.BlockSpec((BLK, D), lambda i: (i, 0)),
            pl.BlockSpec((R, BLK, D), lambda i: (0, i, 0)),
            pl.BlockSpec((R, BLK), lambda i: (0, i)),
            pl.BlockSpec((5 * D, D), lambda i: (0, 0)),
            pl.BlockSpec((1, D), lambda i: (0, 0)),
            pl.BlockSpec((1, D), lambda i: (0, 0)),
        ],
        out_specs=pl.BlockSpec((BLK, D), lambda i: (i, 0)),
        out_shape=jax.ShapeDtypeStruct((NP, D), jnp.float32),
    )(xp, accf, cnt, wstack, gs, bs)


def kernel(x, edge_index, edge_type, W0, root0, b0, bn_gamma, bn_beta,
           bn_mean, bn_var, W1, root1, b1):
    f32 = jnp.float32
    src = edge_index[0].astype(jnp.int32)
    dst = edge_index[1].astype(jnp.int32)
    et = edge_type.astype(jnp.int32)

    ar = jnp.arange(EP - E, dtype=jnp.int32)
    gflat = jnp.concatenate([src, ar % N]).reshape(16 * G, KG, 128)
    sflat = jnp.concatenate([et * NP + dst, R * NP + (ar % 16)])
    pidx = jnp.concatenate([gflat, sflat.reshape(16 * G, KG, 128)], axis=1)
    cidx = sflat.reshape(32, GC, KGC, 128)

    zeros = jnp.zeros((SLICE, CW), f32)
    zeros16 = jnp.zeros((SLICE, 16), f32)
    ones = jnp.ones((128, 16), f32)

    xp = jnp.pad(x, ((0, NP - N), (0, 0)))

    gp = bn_gamma / jnp.sqrt(bn_var + BN_EPS)
    gs0 = gp.reshape(1, D)
    bs0 = ((b0 - bn_mean) * gp + bn_beta).reshape(1, D)
    gs1 = jnp.ones((1, D), f32)
    bs1 = b1.reshape(1, D)
    wstack0 = jnp.concatenate([root0] + [W0[r] for r in range(R)], axis=0)
    wstack1 = jnp.concatenate([root1] + [W1[r] for r in range(R)], axis=0)

    cnt2 = _sc_count_pass(cidx, zeros16, ones)
    cnt = (cnt2[0, :R * NP, 0] + cnt2[1, :R * NP, 0]).reshape(R, NP)
    acc0 = _sc_edge_pass(xp, pidx, zeros)
    h = _tc_dense(xp, acc0[:R * NP].reshape(R, NP, D), cnt,
                  wstack0, gs0, bs0, relu=True)
    acc1 = _sc_edge_pass(h, pidx, zeros)
    out = _tc_dense(h, acc1[:R * NP].reshape(R, NP, D), cnt,
                    wstack1, gs1, bs1, relu=False)
    return out[:N]
